# baseline (speedup 1.0000x reference)
import functools

import jax
import jax.numpy as jnp
from jax import lax
from jax.experimental import pallas as pl
from jax.experimental.pallas import tpu as pltpu
from jax.experimental.pallas import tpu_sc as plsc


def _sc_pool_sum(x3d, table, H, num_cores, num_subcores, lanes):
    NW, BPW, P = x3d.shape
    V, D = table.shape
    DV = D // lanes
    HP = (H + 7) // 8 * 8
    NBUF = 4

    mesh = plsc.VectorSubcoreMesh(core_axis_name="c", subcore_axis_name="s")

    @functools.partial(
        pl.kernel,
        out_type=jax.ShapeDtypeStruct((NW * BPW, D), jnp.float32),
        mesh=mesh,
        scratch_types=[
            pltpu.VMEM((BPW, P), jnp.int32),
            pltpu.VMEM((NBUF, HP, D), jnp.float32),
            pltpu.VMEM((BPW, D), jnp.float32),
            [pltpu.SemaphoreType.DMA] * NBUF,
            pltpu.SemaphoreType.DMA,
        ],
        compiler_params=pltpu.CompilerParams(use_tc_tiling_on_sc=False),
    )
    def k(x_hbm, tab_hbm, out_hbm, idx_v, rows_v, pool_v, sems, sem_o):
        wid = lax.axis_index("s") * num_cores + lax.axis_index("c")
        pltpu.sync_copy(x_hbm.at[wid], idx_v)

        def fire(g, b):
            pltpu.async_copy(
                tab_hbm.at[idx_v.at[g, pl.ds(0, HP)]], rows_v.at[b], sems[b])

        def drain(b):
            pltpu.make_async_copy(
                tab_hbm.at[idx_v.at[0, pl.ds(0, HP)]], rows_v.at[b], sems[b]
            ).wait()

        def accumulate(g, b):
            def acc_body(r, accs):
                return tuple(
                    accs[d] + rows_v[b, r, pl.ds(d * lanes, lanes)]
                    for d in range(DV)
                )
            init = tuple(
                rows_v[b, 0, pl.ds(d * lanes, lanes)] for d in range(DV)
            )
            accs = lax.fori_loop(1, H, acc_body, init, unroll=7)
            for d in range(DV):
                pool_v[g, pl.ds(d * lanes, lanes)] = accs[d]

        for b in range(NBUF):
            fire(b, b)

        @pl.loop(0, BPW - NBUF, step=NBUF)
        def main(g0):
            for b in range(NBUF):
                drain(b)
                accumulate(g0 + b, b)
                fire(g0 + b + NBUF, b)

        for b in range(NBUF):
            drain(b)
            accumulate(BPW - NBUF + b, b)

        base = wid * BPW
        pltpu.async_copy(pool_v, out_hbm.at[pl.ds(base, BPW), :], sem_o).wait()

    return k(x3d, table)


def _tc_mlp(pooled_sum, W1, b1, W2, b2, inv_h):
    B, D = pooled_sum.shape
    F = W1.shape[0]
    NO = W2.shape[0]
    bm = 2048

    def body(p_ref, w1_ref, b1_ref, w2_ref, b2_ref, o_ref):
        p = p_ref[...] * inv_h
        h = lax.dot_general(
            p, w1_ref[...], (((1,), (1,)), ((), ())),
            preferred_element_type=jnp.float32,
        ) + b1_ref[...]
        h = jnp.maximum(h, 0.0)
        o = lax.dot_general(
            h, w2_ref[...], (((1,), (1,)), ((), ())),
            preferred_element_type=jnp.float32,
        ) + b2_ref[0, 0]
        o_ref[...] = jax.nn.sigmoid(o)

    return pl.pallas_call(
        body,
        grid=(B // bm,),
        in_specs=[
            pl.BlockSpec((bm, D), lambda i: (i, 0)),
            pl.BlockSpec((F, D), lambda i: (0, 0)),
            pl.BlockSpec((1, F), lambda i: (0, 0)),
            pl.BlockSpec((NO, F), lambda i: (0, 0)),
            pl.BlockSpec(memory_space=pltpu.SMEM),
        ],
        out_specs=pl.BlockSpec((bm, NO), lambda i: (i, 0)),
        out_shape=jax.ShapeDtypeStruct((B, NO), jnp.float32),
    )(pooled_sum, W1, b1.reshape(1, F), W2, b2.reshape(1, 1))


def kernel(x, table, W1, b1, W2, b2):
    B, H = x.shape
    V, D = table.shape
    info = plsc.get_sparse_core_info()
    NW = info.num_cores * info.num_subcores
    BPW = B // NW
    P = 128
    x3d = jnp.pad(x, ((0, 0), (0, P - H))).reshape(NW, BPW, P)
    pooled_sum = _sc_pool_sum(x3d, table, H, info.num_cores,
                              info.num_subcores, info.num_lanes)
    W2p = jnp.pad(W2, ((0, 8 - W2.shape[0]), (0, 0)))
    out = _tc_mlp(pooled_sum, W1, b1, W2p, b2, 1.0 / H)
    return out[:, : W2.shape[0]]


# speedup vs baseline: 3.8985x; 3.8985x over previous
import functools

import jax
import jax.numpy as jnp
from jax import lax
from jax.experimental import pallas as pl
from jax.experimental.pallas import tpu as pltpu
from jax.experimental.pallas import tpu_sc as plsc


def _sc_pool_sum(x2d, table, H, num_cores, num_subcores, lanes):
    B, P = x2d.shape
    NW = num_cores * num_subcores
    BPW = B // NW
    V, D = table.shape
    DV = D // lanes
    HP = (H + 7) // 8 * 8
    NBUF = 4

    mesh = plsc.VectorSubcoreMesh(core_axis_name="c", subcore_axis_name="s")

    @functools.partial(
        pl.kernel,
        out_type=jax.ShapeDtypeStruct((B, D), jnp.float32),
        mesh=mesh,
        scratch_types=[
            pltpu.VMEM((BPW, P), jnp.int32),
            pltpu.VMEM((NBUF, HP, D), jnp.float32),
            pltpu.VMEM((BPW, D), jnp.float32),
            [pltpu.SemaphoreType.DMA] * NBUF,
            pltpu.SemaphoreType.DMA,
        ],
        compiler_params=pltpu.CompilerParams(use_tc_tiling_on_sc=False),
    )
    def k(x_hbm, tab_hbm, out_hbm, idx_v, rows_v, pool_v, sems, sem_o):
        wid = lax.axis_index("s") * num_cores + lax.axis_index("c")
        base = wid * BPW
        pltpu.sync_copy(x_hbm.at[pl.ds(base, BPW), :], idx_v)

        def fire(g, b):
            pltpu.async_copy(
                tab_hbm.at[idx_v.at[g, pl.ds(0, HP)]], rows_v.at[b], sems[b])

        def drain(b):
            pltpu.make_async_copy(
                tab_hbm.at[idx_v.at[0, pl.ds(0, HP)]], rows_v.at[b], sems[b]
            ).wait()

        def accumulate(g, b):
            def acc_body(r, accs):
                return tuple(
                    accs[d] + rows_v[b, r, pl.ds(d * lanes, lanes)]
                    for d in range(DV)
                )
            init = tuple(
                rows_v[b, 0, pl.ds(d * lanes, lanes)] for d in range(DV)
            )
            accs = lax.fori_loop(1, H, acc_body, init, unroll=7)
            for d in range(DV):
                pool_v[g, pl.ds(d * lanes, lanes)] = accs[d]

        for b in range(NBUF):
            fire(b, b)

        @pl.loop(0, BPW - NBUF, step=NBUF)
        def main(g0):
            for b in range(NBUF):
                drain(b)
                accumulate(g0 + b, b)
                fire(g0 + b + NBUF, b)

        for b in range(NBUF):
            drain(b)
            accumulate(BPW - NBUF + b, b)

        pltpu.async_copy(pool_v, out_hbm.at[pl.ds(base, BPW), :], sem_o).wait()

    return k(x2d, table)


def _tc_mlp(pooled_sum, W1, b1, W2, b2, inv_h):
    B, D = pooled_sum.shape
    F = W1.shape[0]
    NO = W2.shape[0]
    bm = 2048

    def body(p_ref, w1_ref, b1_ref, w2_ref, b2_ref, o_ref):
        p = p_ref[...] * inv_h
        h = lax.dot_general(
            p, w1_ref[...], (((1,), (1,)), ((), ())),
            preferred_element_type=jnp.float32,
        ) + b1_ref[...]
        h = jnp.maximum(h, 0.0)
        o = lax.dot_general(
            h, w2_ref[...], (((1,), (1,)), ((), ())),
            preferred_element_type=jnp.float32,
        ) + b2_ref[0, 0]
        o_ref[...] = jax.nn.sigmoid(o)

    return pl.pallas_call(
        body,
        grid=(B // bm,),
        in_specs=[
            pl.BlockSpec((bm, D), lambda i: (i, 0)),
            pl.BlockSpec((F, D), lambda i: (0, 0)),
            pl.BlockSpec((1, F), lambda i: (0, 0)),
            pl.BlockSpec((NO, F), lambda i: (0, 0)),
            pl.BlockSpec(memory_space=pltpu.SMEM),
        ],
        out_specs=pl.BlockSpec((bm, NO), lambda i: (i, 0)),
        out_shape=jax.ShapeDtypeStruct((B, NO), jnp.float32),
    )(pooled_sum, W1, b1.reshape(1, F), W2, b2.reshape(1, 1))


def _tc_depad(table, VP, bn):
    VPc, D = table.shape
    bm = bn // 2
    nb = VP // bn
    tT = table.T
    eye = jnp.eye(D, dtype=table.dtype)

    def body(a_ref, e_ref, o_ref):
        cdims = (((0,), (0,)), ((), ()))
        blk = a_ref[...]
        r1 = lax.dot_general(blk[:, :bm], e_ref[...], cdims,
                             preferred_element_type=jnp.float32)
        r2 = lax.dot_general(blk[:, bm:], e_ref[...], cdims,
                             preferred_element_type=jnp.float32)
        o_ref[...] = jnp.concatenate([r1, r2], axis=1)

    return pl.pallas_call(
        body,
        grid=(nb,),
        in_specs=[
            pl.BlockSpec((D, bn), lambda i: (0, i)),
            pl.BlockSpec((D, D), lambda i: (0, 0)),
        ],
        out_specs=pl.BlockSpec((bm, 2 * D), lambda i: (i, 0)),
        out_shape=jax.ShapeDtypeStruct((VP // 2, 2 * D), jnp.float32),
    )(tT, eye)

def kernel(x, table, W1, b1, W2, b2):
    B, H = x.shape
    V, D = table.shape
    info = plsc.get_sparse_core_info()
    NW = info.num_cores * info.num_subcores
    BPW = B // NW
    P = 128
    HP = (H + 7) // 8 * 8
    bn = 4096
    VP = (V + bn - 1) // bn * bn
    tp = jnp.pad(table, ((0, VP - V), (0, 0)))
    xe = (x & ~(bn - 1)) + 2 * (x & (bn // 2 - 1)) + ((x >> 11) & 1)
    x2d = jnp.concatenate(
        [xe, xe[:, : HP - H], jnp.zeros((B, P - HP), jnp.int32)], axis=1
    )
    tlin = _tc_depad(tp, VP, bn).reshape(VP, D)
    pooled_sum = _sc_pool_sum(x2d, tlin, H, info.num_cores,
                              info.num_subcores, info.num_lanes)
    W2p = jnp.pad(W2, ((0, 8 - W2.shape[0]), (0, 0)))
    out = _tc_mlp(pooled_sum, W1, b1, W2p, b2, 1.0 / H)
    return out[:, : W2.shape[0]]


# speedup vs baseline: 5.1414x; 1.3188x over previous
import functools

import jax
import jax.numpy as jnp
from jax import lax
from jax.experimental import pallas as pl
from jax.experimental.pallas import tpu as pltpu
from jax.experimental.pallas import tpu_sc as plsc


def _sc_pool_sum(x2d, table, H, num_cores, num_subcores, lanes):
    B, P = x2d.shape
    NW = num_cores * num_subcores
    BPW = B // NW
    V, D = table.shape
    DV = D // lanes
    HP = (H + 7) // 8 * 8
    NBUF = 4

    mesh = plsc.VectorSubcoreMesh(core_axis_name="c", subcore_axis_name="s")

    @functools.partial(
        pl.kernel,
        out_type=jax.ShapeDtypeStruct((B, D), jnp.float32),
        mesh=mesh,
        scratch_types=[
            pltpu.VMEM((BPW, P), jnp.int32),
            pltpu.VMEM((NBUF, HP, D), jnp.float32),
            pltpu.VMEM((BPW, D), jnp.float32),
            [pltpu.SemaphoreType.DMA] * NBUF,
            pltpu.SemaphoreType.DMA,
        ],
        compiler_params=pltpu.CompilerParams(use_tc_tiling_on_sc=False),
    )
    def k(x_hbm, tab_hbm, out_hbm, idx_v, rows_v, pool_v, sems, sem_o):
        wid = lax.axis_index("s") * num_cores + lax.axis_index("c")
        base = wid * BPW
        pltpu.sync_copy(x_hbm.at[pl.ds(base, BPW), :], idx_v)

        def fire(g, b):
            pltpu.async_copy(
                tab_hbm.at[idx_v.at[g, pl.ds(0, HP)]], rows_v.at[b], sems[b])

        def drain(b):
            pltpu.make_async_copy(
                tab_hbm.at[idx_v.at[0, pl.ds(0, HP)]], rows_v.at[b], sems[b]
            ).wait()

        def accumulate(g, b):
            def acc_body(r, accs):
                return tuple(
                    accs[d] + rows_v[b, r, pl.ds(d * lanes, lanes)]
                    for d in range(DV)
                )
            init = tuple(
                rows_v[b, 0, pl.ds(d * lanes, lanes)] for d in range(DV)
            )
            accs = lax.fori_loop(1, H, acc_body, init, unroll=7)
            for d in range(DV):
                pool_v[g, pl.ds(d * lanes, lanes)] = accs[d]

        for b in range(NBUF):
            fire(b, b)

        @pl.loop(0, BPW - NBUF, step=NBUF)
        def main(g0):
            for b in range(NBUF):
                drain(b)
                accumulate(g0 + b, b)
                fire(g0 + b + NBUF, b)

        for b in range(NBUF):
            drain(b)
            accumulate(BPW - NBUF + b, b)

        pltpu.async_copy(pool_v, out_hbm.at[pl.ds(base, BPW), :], sem_o).wait()

    return k(x2d, table)


def _tc_mlp(pooled_sum, W1, b1, W2, b2, inv_h):
    B, D = pooled_sum.shape
    F = W1.shape[0]
    NO = W2.shape[0]
    bm = 2048

    def body(p_ref, w1_ref, b1_ref, w2_ref, b2_ref, o_ref):
        p = p_ref[...] * inv_h
        h = lax.dot_general(
            p, w1_ref[...], (((1,), (1,)), ((), ())),
            preferred_element_type=jnp.float32,
        ) + b1_ref[...]
        h = jnp.maximum(h, 0.0)
        o = lax.dot_general(
            h, w2_ref[...], (((1,), (1,)), ((), ())),
            preferred_element_type=jnp.float32,
        ) + b2_ref[0, 0]
        o_ref[...] = jax.nn.sigmoid(o)

    return pl.pallas_call(
        body,
        grid=(B // bm,),
        in_specs=[
            pl.BlockSpec((bm, D), lambda i: (i, 0)),
            pl.BlockSpec((F, D), lambda i: (0, 0)),
            pl.BlockSpec((1, F), lambda i: (0, 0)),
            pl.BlockSpec((NO, F), lambda i: (0, 0)),
            pl.BlockSpec(memory_space=pltpu.SMEM),
        ],
        out_specs=pl.BlockSpec((bm, NO), lambda i: (i, 0)),
        out_shape=jax.ShapeDtypeStruct((B, NO), jnp.float32),
    )(pooled_sum, W1, b1.reshape(1, F), W2, b2.reshape(1, 1))


def _tc_depad(table, VP, bn):
    bm = bn // 2
    nb = VP // bn
    D = table.shape[1]
    tT = table.T
    eye = jnp.eye(D, dtype=table.dtype)

    def body(a_ref, e_ref, o_ref):
        cdims = (((0,), (0,)), ((), ()))
        blk = a_ref[...]
        r1 = lax.dot_general(blk[:, :bm], e_ref[...], cdims,
                             preferred_element_type=jnp.float32)
        r2 = lax.dot_general(blk[:, bm:], e_ref[...], cdims,
                             preferred_element_type=jnp.float32)
        o_ref[...] = jnp.concatenate([r1, r2], axis=1)

    return pl.pallas_call(
        body,
        grid=(nb,),
        in_specs=[
            pl.BlockSpec((D, bn), lambda i: (0, i)),
            pl.BlockSpec((D, D), lambda i: (0, 0)),
        ],
        out_specs=pl.BlockSpec((bm, 2 * D), lambda i: (i, 0)),
        out_shape=jax.ShapeDtypeStruct((VP // 2, 2 * D), jnp.float32),
    )(tT, eye)

def kernel(x, table, W1, b1, W2, b2):
    B, H = x.shape
    V, D = table.shape
    info = plsc.get_sparse_core_info()
    NW = info.num_cores * info.num_subcores
    BPW = B // NW
    P = 128
    HP = (H + 7) // 8 * 8
    bn = 4096
    VP = (V + bn - 1) // bn * bn
    xe = (x & ~(bn - 1)) + 2 * (x & (bn // 2 - 1)) + ((x >> 11) & 1)
    x2d = jnp.concatenate(
        [xe, xe[:, : HP - H], jnp.zeros((B, P - HP), jnp.int32)], axis=1
    )
    tlin = _tc_depad(table, VP, bn).reshape(VP, D)
    pooled_sum = _sc_pool_sum(x2d, tlin, H, info.num_cores,
                              info.num_subcores, info.num_lanes)
    W2p = jnp.pad(W2, ((0, 8 - W2.shape[0]), (0, 0)))
    out = _tc_mlp(pooled_sum, W1, b1, W2p, b2, 1.0 / H)
    return out[:, : W2.shape[0]]


# speedup vs baseline: 5.9443x; 1.1562x over previous
import functools

import jax
import jax.numpy as jnp
from jax import lax
from jax.experimental import pallas as pl
from jax.experimental.pallas import tpu as pltpu
from jax.experimental.pallas import tpu_sc as plsc


def _sc_pool_sum(x2d, table, H, num_cores, num_subcores, lanes):
    B, P = x2d.shape
    NW = num_cores * num_subcores
    BPW = B // NW
    V, D = table.shape
    DV = D // lanes
    HP = (H + 7) // 8 * 8
    NBUF = 4

    mesh = plsc.VectorSubcoreMesh(core_axis_name="c", subcore_axis_name="s")

    @functools.partial(
        pl.kernel,
        out_type=jax.ShapeDtypeStruct((B, D), jnp.float32),
        mesh=mesh,
        scratch_types=[
            pltpu.VMEM((BPW, P), jnp.int32),
            pltpu.VMEM((NBUF, HP, D), jnp.float32),
            pltpu.VMEM((BPW, D), jnp.float32),
            [pltpu.SemaphoreType.DMA] * NBUF,
            pltpu.SemaphoreType.DMA,
        ],
        compiler_params=pltpu.CompilerParams(use_tc_tiling_on_sc=False),
    )
    def k(x_hbm, tab_hbm, out_hbm, idx_v, rows_v, pool_v, sems, sem_o):
        wid = lax.axis_index("s") * num_cores + lax.axis_index("c")
        base = wid * BPW
        pltpu.sync_copy(x_hbm.at[pl.ds(base, BPW), :], idx_v)

        def fire(g, b):
            pltpu.async_copy(
                tab_hbm.at[idx_v.at[g, pl.ds(0, HP)]], rows_v.at[b], sems[b])

        def drain(b):
            pltpu.make_async_copy(
                tab_hbm.at[idx_v.at[0, pl.ds(0, HP)]], rows_v.at[b], sems[b]
            ).wait()

        def accumulate(g, b):
            def acc_body(r, accs):
                return tuple(
                    accs[d] + rows_v[b, r, pl.ds(d * lanes, lanes)]
                    for d in range(DV)
                )
            init = tuple(
                rows_v[b, 0, pl.ds(d * lanes, lanes)] for d in range(DV)
            )
            accs = lax.fori_loop(1, H, acc_body, init, unroll=7)
            for d in range(DV):
                pool_v[g, pl.ds(d * lanes, lanes)] = accs[d]

        for b in range(NBUF):
            fire(b, b)

        @pl.loop(0, BPW - NBUF, step=NBUF)
        def main(g0):
            for b in range(NBUF):
                drain(b)
                accumulate(g0 + b, b)
                fire(g0 + b + NBUF, b)

        for b in range(NBUF):
            drain(b)
            accumulate(BPW - NBUF + b, b)

        pltpu.async_copy(pool_v, out_hbm.at[pl.ds(base, BPW), :], sem_o).wait()

    return k(x2d, table)


def _tc_mlp(pooled_sum, W1, b1, W2, b2, inv_h):
    B, D = pooled_sum.shape
    F = W1.shape[0]
    NO = W2.shape[0]
    bm = 2048

    def body(p_ref, w1_ref, b1_ref, w2_ref, b2_ref, o_ref):
        p = p_ref[...] * inv_h
        h = lax.dot_general(
            p, w1_ref[...], (((1,), (1,)), ((), ())),
            preferred_element_type=jnp.float32,
        ) + b1_ref[...]
        h = jnp.maximum(h, 0.0)
        o = lax.dot_general(
            h, w2_ref[...], (((1,), (1,)), ((), ())),
            preferred_element_type=jnp.float32,
        ) + b2_ref[0, 0]
        o_ref[...] = jax.nn.sigmoid(o)

    return pl.pallas_call(
        body,
        grid=(B // bm,),
        in_specs=[
            pl.BlockSpec((bm, D), lambda i: (i, 0)),
            pl.BlockSpec((F, D), lambda i: (0, 0)),
            pl.BlockSpec((1, F), lambda i: (0, 0)),
            pl.BlockSpec((NO, F), lambda i: (0, 0)),
            pl.BlockSpec(memory_space=pltpu.SMEM),
        ],
        out_specs=pl.BlockSpec((bm, NO), lambda i: (i, 0)),
        out_shape=jax.ShapeDtypeStruct((B, NO), jnp.float32),
    )(pooled_sum, W1, b1.reshape(1, F), W2, b2.reshape(1, 1))


def _tc_depad(table, VP, bn):
    bm = bn // 2
    nb = VP // bn
    D = table.shape[1]
    tT = table.T
    eye = jnp.eye(D, dtype=table.dtype)

    def body(a_ref, e_ref, o_ref):
        cdims = (((0,), (0,)), ((), ()))
        blk = a_ref[...]
        r1 = lax.dot_general(blk[:, :bm], e_ref[...], cdims,
                             preferred_element_type=jnp.float32)
        r2 = lax.dot_general(blk[:, bm:], e_ref[...], cdims,
                             preferred_element_type=jnp.float32)
        o_ref[...] = jnp.concatenate([r1, r2], axis=1)

    return pl.pallas_call(
        body,
        grid=(nb,),
        in_specs=[
            pl.BlockSpec((D, bn), lambda i: (0, i)),
            pl.BlockSpec((D, D), lambda i: (0, 0)),
        ],
        out_specs=pl.BlockSpec((bm, 2 * D), lambda i: (i, 0)),
        out_shape=jax.ShapeDtypeStruct((VP // 2, 2 * D), jnp.float32),
    )(tT, eye)

def kernel(x, table, W1, b1, W2, b2):
    B, H = x.shape
    V, D = table.shape
    info = plsc.get_sparse_core_info()
    NW = info.num_cores * info.num_subcores
    BPW = B // NW
    P = 128
    HP = (H + 7) // 8 * 8
    bn = 8192
    VP = (V + bn - 1) // bn * bn
    xe = (x & ~(bn - 1)) + 2 * (x & (bn // 2 - 1)) + ((x >> 12) & 1)
    x2d = jnp.concatenate(
        [xe, xe[:, : HP - H], jnp.zeros((B, P - HP), jnp.int32)], axis=1
    )
    tlin = _tc_depad(table, VP, bn).reshape(VP, D)
    pooled_sum = _sc_pool_sum(x2d, tlin, H, info.num_cores,
                              info.num_subcores, info.num_lanes)
    W2p = jnp.pad(W2, ((0, 8 - W2.shape[0]), (0, 0)))
    out = _tc_mlp(pooled_sum, W1, b1, W2p, b2, 1.0 / H)
    return out[:, : W2.shape[0]]


# speedup vs baseline: 6.3960x; 1.0760x over previous
import functools

import jax
import jax.numpy as jnp
from jax import lax
from jax.experimental import pallas as pl
from jax.experimental.pallas import tpu as pltpu
from jax.experimental.pallas import tpu_sc as plsc


def _sc_pool_sum(x2d, table, H, num_cores, num_subcores, lanes):
    B, P = x2d.shape
    NW = num_cores * num_subcores
    BPW = B // NW
    V, D = table.shape
    DV = D // lanes
    HP = (H + 7) // 8 * 8
    NBUF = 4

    mesh = plsc.VectorSubcoreMesh(core_axis_name="c", subcore_axis_name="s")

    @functools.partial(
        pl.kernel,
        out_type=jax.ShapeDtypeStruct((B, D), jnp.float32),
        mesh=mesh,
        scratch_types=[
            pltpu.VMEM((BPW, P), jnp.int32),
            pltpu.VMEM((NBUF, HP, D), jnp.float32),
            pltpu.VMEM((BPW, D), jnp.float32),
            [pltpu.SemaphoreType.DMA] * NBUF,
            pltpu.SemaphoreType.DMA,
        ],
        compiler_params=pltpu.CompilerParams(use_tc_tiling_on_sc=False),
    )
    def k(x_hbm, tab_hbm, out_hbm, idx_v, rows_v, pool_v, sems, sem_o):
        wid = lax.axis_index("s") * num_cores + lax.axis_index("c")
        base = wid * BPW
        pltpu.sync_copy(x_hbm.at[pl.ds(base, BPW), :], idx_v)

        def fire(g, b):
            pltpu.async_copy(
                tab_hbm.at[idx_v.at[g, pl.ds(0, HP)]], rows_v.at[b], sems[b])

        def drain(b):
            pltpu.make_async_copy(
                tab_hbm.at[idx_v.at[0, pl.ds(0, HP)]], rows_v.at[b], sems[b]
            ).wait()

        def accumulate(g, b):
            def acc_body(r, accs):
                return tuple(
                    accs[d] + rows_v[b, r, pl.ds(d * lanes, lanes)]
                    for d in range(DV)
                )
            init = tuple(
                rows_v[b, 0, pl.ds(d * lanes, lanes)] for d in range(DV)
            )
            accs = lax.fori_loop(1, H, acc_body, init, unroll=7)
            for d in range(DV):
                pool_v[g, pl.ds(d * lanes, lanes)] = accs[d]

        for b in range(NBUF):
            fire(b, b)

        @pl.loop(0, BPW - NBUF, step=NBUF)
        def main(g0):
            for b in range(NBUF):
                drain(b)
                accumulate(g0 + b, b)
                fire(g0 + b + NBUF, b)

        for b in range(NBUF):
            drain(b)
            accumulate(BPW - NBUF + b, b)

        pltpu.async_copy(pool_v, out_hbm.at[pl.ds(base, BPW), :], sem_o).wait()

    return k(x2d, table)


def _tc_mlp(pooled_sum, W1, b1, W2, b2, inv_h):
    B, D = pooled_sum.shape
    F = W1.shape[0]
    NO = W2.shape[0]
    bm = 2048

    def body(p_ref, w1_ref, b1_ref, w2_ref, b2_ref, o_ref):
        p = p_ref[...] * inv_h
        h = lax.dot_general(
            p, w1_ref[...], (((1,), (1,)), ((), ())),
            preferred_element_type=jnp.float32,
        ) + b1_ref[...]
        h = jnp.maximum(h, 0.0)
        o = lax.dot_general(
            h, w2_ref[...], (((1,), (1,)), ((), ())),
            preferred_element_type=jnp.float32,
        ) + b2_ref[0, 0]
        o_ref[...] = jax.nn.sigmoid(o)

    return pl.pallas_call(
        body,
        grid=(B // bm,),
        in_specs=[
            pl.BlockSpec((bm, D), lambda i: (i, 0)),
            pl.BlockSpec((F, D), lambda i: (0, 0)),
            pl.BlockSpec((1, F), lambda i: (0, 0)),
            pl.BlockSpec((NO, F), lambda i: (0, 0)),
            pl.BlockSpec(memory_space=pltpu.SMEM),
        ],
        out_specs=pl.BlockSpec((bm, NO), lambda i: (i, 0)),
        out_shape=jax.ShapeDtypeStruct((B, NO), jnp.float32),
    )(pooled_sum, W1, b1.reshape(1, F), W2, b2.reshape(1, 1))


def _tc_depad(table, VP, bn):
    bm = bn // 2
    nb = VP // bn
    D = table.shape[1]
    tT = table.T
    eye = jnp.eye(D, dtype=table.dtype)

    def body(a_ref, e_ref, o_ref):
        cdims = (((0,), (0,)), ((), ()))
        blk = a_ref[...]
        r1 = lax.dot_general(blk[:, :bm], e_ref[...], cdims,
                             preferred_element_type=jnp.float32)
        r2 = lax.dot_general(blk[:, bm:], e_ref[...], cdims,
                             preferred_element_type=jnp.float32)
        o_ref[...] = jnp.concatenate([r1, r2], axis=1)

    return pl.pallas_call(
        body,
        grid=(nb,),
        in_specs=[
            pl.BlockSpec((D, bn), lambda i: (0, i)),
            pl.BlockSpec((D, D), lambda i: (0, 0)),
        ],
        out_specs=pl.BlockSpec((bm, 2 * D), lambda i: (i, 0)),
        out_shape=jax.ShapeDtypeStruct((VP // 2, 2 * D), jnp.float32),
    )(tT, eye)

def kernel(x, table, W1, b1, W2, b2):
    B, H = x.shape
    V, D = table.shape
    info = plsc.get_sparse_core_info()
    NW = info.num_cores * info.num_subcores
    BPW = B // NW
    P = 128
    HP = (H + 7) // 8 * 8
    bn = 16384
    VP = (V + bn - 1) // bn * bn
    xe = (x & ~(bn - 1)) + 2 * (x & (bn // 2 - 1)) + ((x >> 13) & 1)
    x2d = jnp.concatenate(
        [xe, xe[:, : HP - H], jnp.zeros((B, P - HP), jnp.int32)], axis=1
    )
    tlin = _tc_depad(table, VP, bn).reshape(VP, D)
    pooled_sum = _sc_pool_sum(x2d, tlin, H, info.num_cores,
                              info.num_subcores, info.num_lanes)
    W2p = jnp.pad(W2, ((0, 8 - W2.shape[0]), (0, 0)))
    out = _tc_mlp(pooled_sum, W1, b1, W2p, b2, 1.0 / H)
    return out[:, : W2.shape[0]]


# speedup vs baseline: 6.6334x; 1.0371x over previous
import functools

import jax
import jax.numpy as jnp
from jax import lax
from jax.experimental import pallas as pl
from jax.experimental.pallas import tpu as pltpu
from jax.experimental.pallas import tpu_sc as plsc


def _sc_pool_sum(x2d, table, H, num_cores, num_subcores, lanes):
    B, P = x2d.shape
    NW = num_cores * num_subcores
    BPW = B // NW
    V, D = table.shape
    DV = D // lanes
    HP = (H + 7) // 8 * 8
    NBUF = 4

    mesh = plsc.VectorSubcoreMesh(core_axis_name="c", subcore_axis_name="s")

    @functools.partial(
        pl.kernel,
        out_type=jax.ShapeDtypeStruct((B, D), jnp.float32),
        mesh=mesh,
        scratch_types=[
            pltpu.VMEM((BPW, P), jnp.int32),
            pltpu.VMEM((NBUF, HP, D), jnp.float32),
            pltpu.VMEM((BPW, D), jnp.float32),
            [pltpu.SemaphoreType.DMA] * NBUF,
            pltpu.SemaphoreType.DMA,
        ],
        compiler_params=pltpu.CompilerParams(use_tc_tiling_on_sc=False),
    )
    def k(x_hbm, tab_hbm, out_hbm, idx_v, rows_v, pool_v, sems, sem_o):
        wid = lax.axis_index("s") * num_cores + lax.axis_index("c")
        base = wid * BPW
        pltpu.sync_copy(x_hbm.at[pl.ds(base, BPW), :], idx_v)

        def fire(g, b):
            pltpu.async_copy(
                tab_hbm.at[idx_v.at[g, pl.ds(0, HP)]], rows_v.at[b], sems[b])

        def drain(b):
            pltpu.make_async_copy(
                tab_hbm.at[idx_v.at[0, pl.ds(0, HP)]], rows_v.at[b], sems[b]
            ).wait()

        def accumulate(g, b):
            def acc_body(r, accs):
                return tuple(
                    accs[d] + rows_v[b, r, pl.ds(d * lanes, lanes)]
                    for d in range(DV)
                )
            init = tuple(
                rows_v[b, 0, pl.ds(d * lanes, lanes)] for d in range(DV)
            )
            accs = lax.fori_loop(1, H, acc_body, init, unroll=7)
            for d in range(DV):
                pool_v[g, pl.ds(d * lanes, lanes)] = accs[d]

        for b in range(NBUF):
            fire(b, b)

        @pl.loop(0, BPW - NBUF, step=NBUF)
        def main(g0):
            for b in range(NBUF):
                drain(b)
                accumulate(g0 + b, b)
                fire(g0 + b + NBUF, b)

        for b in range(NBUF):
            drain(b)
            accumulate(BPW - NBUF + b, b)

        pltpu.async_copy(pool_v, out_hbm.at[pl.ds(base, BPW), :], sem_o).wait()

    return k(x2d, table)


def _tc_mlp(pooled_sum, W1, b1, W2, b2, inv_h):
    B, D = pooled_sum.shape
    F = W1.shape[0]
    NO = W2.shape[0]
    bm = 2048

    def body(p_ref, w1_ref, b1_ref, w2_ref, b2_ref, o_ref):
        p = p_ref[...] * inv_h
        h = lax.dot_general(
            p, w1_ref[...], (((1,), (1,)), ((), ())),
            preferred_element_type=jnp.float32,
        ) + b1_ref[...]
        h = jnp.maximum(h, 0.0)
        o = lax.dot_general(
            h, w2_ref[...], (((1,), (1,)), ((), ())),
            preferred_element_type=jnp.float32,
        ) + b2_ref[0, 0]
        o_ref[...] = jax.nn.sigmoid(o)

    return pl.pallas_call(
        body,
        grid=(B // bm,),
        in_specs=[
            pl.BlockSpec((bm, D), lambda i: (i, 0)),
            pl.BlockSpec((F, D), lambda i: (0, 0)),
            pl.BlockSpec((1, F), lambda i: (0, 0)),
            pl.BlockSpec((NO, F), lambda i: (0, 0)),
            pl.BlockSpec(memory_space=pltpu.SMEM),
        ],
        out_specs=pl.BlockSpec((bm, NO), lambda i: (i, 0)),
        out_shape=jax.ShapeDtypeStruct((B, NO), jnp.float32),
    )(pooled_sum, W1, b1.reshape(1, F), W2, b2.reshape(1, 1))


def _tc_depad(table, VP, bn):
    bm = bn // 2
    nb = VP // bn
    D = table.shape[1]
    tT = table.T
    eye = jnp.eye(D, dtype=table.dtype)

    def body(a_ref, e_ref, o_ref):
        cdims = (((0,), (0,)), ((), ()))
        blk = a_ref[...]
        r1 = lax.dot_general(blk[:, :bm], e_ref[...], cdims,
                             preferred_element_type=jnp.float32)
        r2 = lax.dot_general(blk[:, bm:], e_ref[...], cdims,
                             preferred_element_type=jnp.float32)
        o_ref[...] = jnp.concatenate([r1, r2], axis=1)

    return pl.pallas_call(
        body,
        grid=(nb,),
        in_specs=[
            pl.BlockSpec((D, bn), lambda i: (0, i)),
            pl.BlockSpec((D, D), lambda i: (0, 0)),
        ],
        out_specs=pl.BlockSpec((bm, 2 * D), lambda i: (i, 0)),
        out_shape=jax.ShapeDtypeStruct((VP // 2, 2 * D), jnp.float32),
    )(tT, eye)

def kernel(x, table, W1, b1, W2, b2):
    B, H = x.shape
    V, D = table.shape
    info = plsc.get_sparse_core_info()
    NW = info.num_cores * info.num_subcores
    BPW = B // NW
    P = 128
    HP = (H + 7) // 8 * 8
    bn = 32768
    VP = (V + bn - 1) // bn * bn
    xe = (x & ~(bn - 1)) + 2 * (x & (bn // 2 - 1)) + ((x >> 14) & 1)
    x2d = jnp.concatenate(
        [xe, xe[:, : HP - H], jnp.zeros((B, P - HP), jnp.int32)], axis=1
    )
    tlin = _tc_depad(table, VP, bn).reshape(VP, D)
    pooled_sum = _sc_pool_sum(x2d, tlin, H, info.num_cores,
                              info.num_subcores, info.num_lanes)
    W2p = jnp.pad(W2, ((0, 8 - W2.shape[0]), (0, 0)))
    out = _tc_mlp(pooled_sum, W1, b1, W2p, b2, 1.0 / H)
    return out[:, : W2.shape[0]]
